# baseline (speedup 1.0000x reference)
import functools
import math

import jax
import jax.numpy as jnp
from jax import lax
from jax.experimental import pallas as pl
from jax.experimental.pallas import tpu as pltpu
from jax.experimental.pallas import tpu_sc as plsc

D_MODEL = 64
SCALE = math.sqrt(D_MODEL)
LANES = 16

_NC = 2
_NS = 16
_NW = _NC * _NS


def _make_kernel(S0, S1):
    NB = S0 // 128
    n_units = S1 * NB
    assert n_units % _NW == 0
    u_per_w = n_units // _NW
    b_per_w = u_per_w * 128

    mesh = plsc.VectorSubcoreMesh(core_axis_name="c", subcore_axis_name="s")

    @functools.partial(
        pl.kernel,
        mesh=mesh,
        out_type=jax.ShapeDtypeStruct((S1, 8, NB, 8, 128), jnp.float32),
        compiler_params=pltpu.CompilerParams(
            use_tc_tiling_on_sc=False, needs_layout_passes=False
        ),
        scratch_types=[
            pltpu.VMEM((b_per_w,), jnp.int32),
            pltpu.VMEM((128, D_MODEL), jnp.float32),
            pltpu.VMEM((8, 8, 128), jnp.float32),
            pltpu.SemaphoreType.DMA,
        ],
    )
    def gather_scale(idx_hbm, table_hbm, out_hbm, idx_v, rows, tbuf, sem):
        wid = lax.axis_index("s") * _NC + lax.axis_index("c")
        ubase = wid * u_per_w
        pltpu.sync_copy(idx_hbm.at[pl.ds(ubase * 128, b_per_w)], idx_v)

        iota = lax.iota(jnp.int32, LANES)

        def unit_body(k, carry):
            u = ubase + k
            s1 = u // NB
            b = u % NB

            pltpu.async_copy(
                table_hbm.at[idx_v.at[pl.ds(k * 128, 128)]], rows, sem
            ).wait()

            def j_body(j, jcarry):
                a = j // 8
                r = j % 8
                jvec = jnp.full((LANES,), j, dtype=jnp.int32)
                for cc in range(128 // LANES):
                    v = plsc.load_gather(rows, [iota + cc * LANES, jvec])
                    tbuf[a, r, pl.ds(cc * LANES, LANES)] = v * SCALE
                return jcarry

            lax.fori_loop(0, D_MODEL, j_body, 0, unroll=2)

            for a in range(8):
                pltpu.sync_copy(tbuf.at[a], out_hbm.at[s1, a, b])
            return carry

        lax.fori_loop(0, u_per_w, unit_body, 0)

    return gather_scale


def kernel(x, table):
    S0, S1 = x.shape
    xt = jnp.swapaxes(x, 0, 1)
    idx = xt.reshape(S0 * S1).astype(jnp.int32)
    out5d = _make_kernel(S0, S1)(idx, table)
    return out5d.transpose(2, 4, 0, 1, 3).reshape(S0, S1, D_MODEL)


# speedup vs baseline: 1.3346x; 1.3346x over previous
import functools
import math

import jax
import jax.numpy as jnp
from jax import lax
from jax.experimental import pallas as pl
from jax.experimental.pallas import tpu as pltpu
from jax.experimental.pallas import tpu_sc as plsc

D_MODEL = 64
SCALE = math.sqrt(D_MODEL)
LANES = 16

_NC = 2
_NS = 16
_NW = _NC * _NS
_G = 2
_U = 128 * _G
_NSLOT = 3


def _make_kernel(S0, S1):
    NB = S0 // 128
    n_units = S1 * NB // _G
    assert n_units % _NW == 0
    u_per_w = n_units // _NW
    b_per_w = u_per_w * _U
    upp = NB // _G
    out_flat = S1 * D_MODEL * S0

    mesh = plsc.VectorSubcoreMesh(core_axis_name="c", subcore_axis_name="s")

    @functools.partial(
        pl.kernel,
        mesh=mesh,
        out_type=jax.ShapeDtypeStruct((out_flat,), jnp.float32),
        compiler_params=pltpu.CompilerParams(
            use_tc_tiling_on_sc=False, needs_layout_passes=False
        ),
        scratch_types=[
            pltpu.VMEM((b_per_w,), jnp.int32),
        ]
        + [pltpu.VMEM((_U, D_MODEL), jnp.float32) for _ in range(_NSLOT)]
        + [pltpu.VMEM((_U * D_MODEL,), jnp.float32) for _ in range(_NSLOT)]
        + [pltpu.SemaphoreType.DMA for _ in range(2 * _NSLOT)],
    )
    def gather_scale(idx_hbm, table_hbm, out_hbm, idx_v, *scratch):
        rows = scratch[:_NSLOT]
        tbuf = scratch[_NSLOT : 2 * _NSLOT]
        gsem = scratch[2 * _NSLOT : 3 * _NSLOT]
        ssem = scratch[3 * _NSLOT :]

        wid = lax.axis_index("s") * _NC + lax.axis_index("c")
        ubase = wid * u_per_w
        pltpu.sync_copy(idx_hbm.at[pl.ds(ubase * _U, b_per_w)], idx_v)

        lane = lax.iota(jnp.int32, LANES)
        cbase = (lane >> 3) * (_G * 1024) + ((lane & 7) << 7)

        def gather(k, s):
            return pltpu.make_async_copy(
                table_hbm.at[idx_v.at[pl.ds(k * _U, _U)]], rows[s], gsem[s]
            )

        def out_base(k, a):
            gu = ubase + k
            s1 = gu // upp
            bo = gu % upp
            return s1 * (D_MODEL * S0) + a * (8 * S0) + bo * (_G * 1024)

        def store(k, a, s):
            return pltpu.make_async_copy(
                tbuf[s].at[pl.ds(a * (_G * 1024), _G * 1024)],
                out_hbm.at[pl.ds(out_base(k, a), _G * 1024)],
                ssem[s],
            )

        def transpose_scale(s):
            def ci_body(ci, carry):
                base = ((ci >> 7) << 10) | (ci & 127)
                for jj in range(D_MODEL // LANES):
                    v = rows[s][ci, pl.ds(jj * LANES, LANES)] * SCALE
                    plsc.store_scatter(
                        tbuf[s], [cbase + (base + jj * 4096)], v
                    )
                return carry

            lax.fori_loop(0, _U, ci_body, 0, unroll=4)

        def step(k, s):
            @pl.when(k + 2 < u_per_w)
            def _():
                gather(k + 2, (s + 2) % _NSLOT).start()

            gather(k, s).wait()

            @pl.when(k >= _NSLOT)
            def _():
                for a in range(8):
                    store(k - _NSLOT, a, s).wait()

            transpose_scale(s)
            for a in range(8):
                store(k, a, s).start()

        gather(0, 0).start()
        gather(1, 1).start()

        def outer(i, carry):
            for jj in range(_NSLOT):
                step(i * _NSLOT + jj, jj)
            return carry

        lax.fori_loop(0, u_per_w // _NSLOT, outer, 0)
        for k in range(_NSLOT * (u_per_w // _NSLOT), u_per_w):
            step(k, k % _NSLOT)

        for k in range(u_per_w - _NSLOT, u_per_w):
            for a in range(8):
                store(k, a, k % _NSLOT).wait()

    return gather_scale


def kernel(x, table):
    S0, S1 = x.shape
    xt = jnp.swapaxes(x, 0, 1)
    idx = xt.reshape(S0 * S1).astype(jnp.int32)
    out1d = _make_kernel(S0, S1)(idx, table)
    out5d = out1d.reshape(S1, 8, S0 // 128, 8, 128)
    return out5d.transpose(2, 4, 0, 1, 3).reshape(S0, S1, D_MODEL)


# speedup vs baseline: 1.4284x; 1.0703x over previous
import functools
import math

import jax
import jax.numpy as jnp
from jax import lax
from jax.experimental import pallas as pl
from jax.experimental.pallas import tpu as pltpu
from jax.experimental.pallas import tpu_sc as plsc

D_MODEL = 64
SCALE = math.sqrt(D_MODEL)
LANES = 16

_NC = 2
_NS = 16
_NW = _NC * _NS
_G = 2
_U = 128 * _G
_NRAW = 3
_NTB = 2
_PAD = 65


def _make_kernel(S0, S1):
    NB = S0 // 128
    n_units = S1 * NB // _G
    assert n_units % _NW == 0
    u_per_w = n_units // _NW
    b_per_w = u_per_w * _U
    upp = NB // _G
    out_flat = S1 * D_MODEL * S0
    span = _G * 1024

    mesh = plsc.VectorSubcoreMesh(core_axis_name="c", subcore_axis_name="s")

    @functools.partial(
        pl.kernel,
        mesh=mesh,
        out_type=jax.ShapeDtypeStruct((out_flat,), jnp.float32),
        compiler_params=pltpu.CompilerParams(
            use_tc_tiling_on_sc=False, needs_layout_passes=False
        ),
        scratch_types=[
            pltpu.VMEM((b_per_w,), jnp.int32),
            pltpu.VMEM((_U, _PAD), jnp.float32),
        ]
        + [pltpu.VMEM((_U, D_MODEL), jnp.float32) for _ in range(_NRAW)]
        + [pltpu.VMEM((_U * D_MODEL,), jnp.float32) for _ in range(_NTB)]
        + [pltpu.SemaphoreType.DMA for _ in range(_NRAW + _NTB)],
    )
    def gather_scale(idx_hbm, table_hbm, out_hbm, idx_v, pad, *scratch):
        raw = scratch[:_NRAW]
        tbuf = scratch[_NRAW : _NRAW + _NTB]
        gsem = scratch[_NRAW + _NTB : 2 * _NRAW + _NTB]
        ssem = scratch[2 * _NRAW + _NTB :]

        wid = lax.axis_index("s") * _NC + lax.axis_index("c")
        ubase = wid * u_per_w
        pltpu.sync_copy(idx_hbm.at[pl.ds(ubase * _U, b_per_w)], idx_v)

        lane = lax.iota(jnp.int32, LANES)

        def gather(k, sr):
            return pltpu.make_async_copy(
                table_hbm.at[idx_v.at[pl.ds(k * _U, _U)]], raw[sr], gsem[sr]
            )

        def out_base(k, a):
            gu = ubase + k
            s1 = gu // upp
            bo = gu % upp
            return s1 * (D_MODEL * S0) + a * (8 * S0) + bo * span

        def store(k, a, st):
            return pltpu.make_async_copy(
                tbuf[st].at[pl.ds(a * span, span)],
                out_hbm.at[pl.ds(out_base(k, a), span)],
                ssem[st],
            )

        def pad_scale(sr):
            src = raw[sr]

            def ci_body(ci, carry):
                for jj in range(D_MODEL // LANES):
                    sl = pl.ds(jj * LANES, LANES)
                    pad[ci, sl] = src[ci, sl] * SCALE
                return carry

            lax.fori_loop(0, _U, ci_body, 0, unroll=4)

        def transpose(st):
            tb = tbuf[st]

            def o_body(o, carry):
                a = o >> 4
                bb = (o >> 3) & (_G - 1)
                r = o & 7
                j = (a << 3) | r
                jvec = lane * 0 + j
                row0 = bb << 7
                tpos0 = o << 7
                for cb in range(128 // LANES):
                    v = plsc.load_gather(
                        pad, [lane + (row0 + cb * LANES), jvec]
                    )
                    tb[pl.ds(tpos0 + cb * LANES, LANES)] = v
                return carry

            lax.fori_loop(0, 8 * _G * 8, o_body, 0, unroll=2)

        def step(k, sr, st):
            @pl.when(k + 2 < u_per_w)
            def _():
                gather(k + 2, (sr + 2) % _NRAW).start()

            gather(k, sr).wait()
            pad_scale(sr)

            @pl.when(k >= _NTB)
            def _():
                for a in range(8):
                    store(k - _NTB, a, st).wait()

            transpose(st)
            for a in range(8):
                store(k, a, st).start()

        gather(0, 0).start()
        gather(1, 1).start()

        PERIOD = 6
        n_main = (u_per_w // PERIOD) * PERIOD

        def outer(i, carry):
            for jj in range(PERIOD):
                step(i * PERIOD + jj, jj % _NRAW, jj % _NTB)
            return carry

        lax.fori_loop(0, u_per_w // PERIOD, outer, 0)
        for k in range(n_main, u_per_w):
            step(k, k % _NRAW, k % _NTB)

        for k in range(u_per_w - _NTB, u_per_w):
            for a in range(8):
                store(k, a, k % _NTB).wait()

    return gather_scale


def kernel(x, table):
    S0, S1 = x.shape
    xt = jnp.swapaxes(x, 0, 1)
    idx = xt.reshape(S0 * S1).astype(jnp.int32)
    out1d = _make_kernel(S0, S1)(idx, table)
    out5d = out1d.reshape(S1, 8, S0 // 128, 8, 128)
    return out5d.transpose(2, 4, 0, 1, 3).reshape(S0, S1, D_MODEL)
